# baseline (speedup 1.0000x reference)
import functools

import jax
import jax.numpy as jnp
from jax import lax
from jax.experimental import pallas as pl
from jax.experimental.pallas import tpu as pltpu
from jax.experimental.pallas import tpu_sc as plsc

D = 128
NC = 2
NS = 16
NW = NC * NS
CHUNK = 80
NSLOT = 5
LA = 2
IB = 20


@functools.partial(jax.jit, static_argnames=("n_blocks",))
def _sc_gather(idx4, weights, n_blocks):
    B = NW * n_blocks * IB * CHUNK
    total = n_blocks * IB
    mesh = plsc.VectorSubcoreMesh(core_axis_name="c", subcore_axis_name="s")

    @functools.partial(
        pl.kernel,
        out_type=jax.ShapeDtypeStruct((B, D), jnp.float32),
        mesh=mesh,
        scratch_types=[
            pltpu.VMEM((2, IB, CHUNK), jnp.int32),
            pltpu.VMEM((NSLOT, CHUNK, D), jnp.float32),
            pltpu.VMEM_SHARED((8192, D), jnp.float32),
            pltpu.SemaphoreType.DMA((NSLOT,)),
            pltpu.SemaphoreType.DMA((NSLOT,)),
            pltpu.SemaphoreType.DMA((2,)),
        ],
    )
    def k(idx_hbm, table_hbm, out_hbm, idx_v, rows_v, table_sh, gsem, ssem, isem):
        wid = lax.axis_index("s") * NC + lax.axis_index("c")
        base = wid * (total * CHUNK)

        sid = lax.axis_index("s")
        rows_per_tile = 8192 // NS
        pltpu.sync_copy(
            table_hbm.at[pl.ds(sid * rows_per_tile, rows_per_tile)],
            table_sh.at[pl.ds(sid * rows_per_tile, rows_per_tile)],
        )
        pltpu.sync_copy(idx_hbm.at[wid, 0], idx_v.at[0])
        plsc.subcore_barrier()

        def load_idx(blk, p):
            return pltpu.make_async_copy(
                idx_hbm.at[wid, blk], idx_v.at[p], isem.at[p]
            )

        def gather(p, j, s):
            src = table_hbm if j % 10 == 5 else table_sh
            return pltpu.make_async_copy(
                src.at[idx_v.at[p].at[j]], rows_v.at[s], gsem.at[s]
            )

        def store(g, s):
            off = base + g * CHUNK
            return pltpu.make_async_copy(
                rows_v.at[s], out_hbm.at[pl.ds(off, CHUNK)], ssem.at[s]
            )

        for j in range(LA):
            gather(0, j, j % NSLOT).start()

        def body(k_, _):
            b = lax.rem(k_, 2)

            @pl.when(k_ + 1 < n_blocks)
            def _():
                load_idx(k_ + 1, 1 - b).start()

            for j in range(IB):
                g = k_ * IB + j
                s = j % NSLOT
                gather(b, j, s).wait()
                store(g, s).start()

                if j == IB - LA:
                    @pl.when(k_ + 1 < n_blocks)
                    def _():
                        load_idx(k_ + 1, 1 - b).wait()

                s2 = (j + LA) % NSLOT

                @pl.when(jnp.logical_and(g + LA < total, g + LA >= NSLOT))
                def _():
                    store(g + LA - NSLOT, s2).wait()

                @pl.when(g + LA < total)
                def _():
                    if j < IB - LA:
                        gather(b, j + LA, s2).start()
                    else:
                        gather(1 - b, j + LA - IB, s2).start()

            return 0

        lax.fori_loop(0, n_blocks, body, 0)

        for m in range(total - NSLOT, total):
            store(m, m % NSLOT).wait()

    return k(idx4, weights)


def kernel(detail_pos, weights):
    shape = detail_pos.shape
    flat = detail_pos.reshape(-1).astype(jnp.int32)
    n_blocks = flat.shape[0] // (NW * IB * CHUNK)
    idx4 = flat.reshape(NW, n_blocks, IB, CHUNK)
    out = _sc_gather(idx4, weights.astype(jnp.float32), n_blocks)
    return out.reshape(shape + (weights.shape[-1],))


# speedup vs baseline: 1.1219x; 1.1219x over previous
import functools

import jax
import jax.numpy as jnp
from jax import lax
from jax.experimental import pallas as pl
from jax.experimental.pallas import tpu as pltpu
from jax.experimental.pallas import tpu_sc as plsc

D = 128
NC = 2
NS = 16
NW = NC * NS
CHUNK = 80
NBUF = 4
IB = 40


@functools.partial(jax.jit, static_argnames=("n_blocks",))
def _sc_gather(idx4, weights, n_blocks):
    B = NW * n_blocks * IB * CHUNK
    total = n_blocks * IB
    mesh = plsc.VectorSubcoreMesh(core_axis_name="c", subcore_axis_name="s")

    @functools.partial(
        pl.kernel,
        out_type=jax.ShapeDtypeStruct((B, D), jnp.float32),
        mesh=mesh,
        scratch_types=[
            pltpu.VMEM((2, IB, CHUNK), jnp.int32),
            pltpu.VMEM((NBUF, CHUNK, D), jnp.float32),
            pltpu.VMEM_SHARED((8192, D), jnp.float32),
            pltpu.SemaphoreType.DMA((NBUF,)),
            pltpu.SemaphoreType.DMA((NBUF,)),
            pltpu.SemaphoreType.DMA((2,)),
        ],
    )
    def k(idx_hbm, table_hbm, out_hbm, idx_v, rows_v, table_sh, gsem, ssem, isem):
        wid = lax.axis_index("s") * NC + lax.axis_index("c")
        base = wid * (total * CHUNK)

        sid = lax.axis_index("s")
        rows_per_tile = 8192 // NS
        pltpu.sync_copy(
            table_hbm.at[pl.ds(sid * rows_per_tile, rows_per_tile)],
            table_sh.at[pl.ds(sid * rows_per_tile, rows_per_tile)],
        )
        pltpu.sync_copy(idx_hbm.at[wid, 0], idx_v.at[0])
        plsc.subcore_barrier()

        def load_idx(blk, p):
            return pltpu.make_async_copy(
                idx_hbm.at[wid, blk], idx_v.at[p], isem.at[p]
            )

        def gather(p, j, s):
            return pltpu.make_async_copy(
                table_sh.at[idx_v.at[p].at[j]], rows_v.at[s], gsem.at[s]
            )

        def store(g, s):
            off = base + g * CHUNK
            return pltpu.make_async_copy(
                rows_v.at[s], out_hbm.at[pl.ds(off, CHUNK)], ssem.at[s]
            )

        for j in range(NBUF):
            gather(0, j, j).start()

        def body(k_, _):
            b = lax.rem(k_, 2)

            @pl.when(k_ + 1 < n_blocks)
            def _():
                load_idx(k_ + 1, 1 - b).start()

            for j in range(IB):
                g = k_ * IB + j
                s = j % NBUF
                gather(b, j, s).wait()
                store(g, s).start()

                if j == IB - NBUF:
                    @pl.when(k_ + 1 < n_blocks)
                    def _():
                        load_idx(k_ + 1, 1 - b).wait()

                @pl.when(g + NBUF < total)
                def _():
                    store(g, s).wait()
                    if j < IB - NBUF:
                        gather(b, j + NBUF, s).start()
                    else:
                        gather(1 - b, j + NBUF - IB, s).start()

            return 0

        lax.fori_loop(0, n_blocks, body, 0)

        for s in range(NBUF):
            store(total - NBUF + s, s).wait()

    return k(idx4, weights)


def kernel(detail_pos, weights):
    shape = detail_pos.shape
    flat = detail_pos.reshape(-1).astype(jnp.int32)
    n_blocks = flat.shape[0] // (NW * IB * CHUNK)
    idx4 = flat.reshape(NW, n_blocks, IB, CHUNK)
    out = _sc_gather(idx4, weights.astype(jnp.float32), n_blocks)
    return out.reshape(shape + (weights.shape[-1],))
